# Initial kernel scaffold; baseline (speedup 1.0000x reference)
import jax
import jax.numpy as jnp
from jax.experimental import pallas as pl


def kernel(inputs, buffer):
    raise NotImplementedError("write your pallas kernel here")


# speedup vs baseline: 46.0123x; 46.0123x over previous
import jax
import jax.numpy as jnp
from jax.experimental import pallas as pl

_BLK = 8


def _gather_row_body(row_in_block, src_ref, out_ref):
    out_ref[...] = src_ref[row_in_block, :][None, :]


def kernel(inputs, buffer):
    buffer_size = buffer.shape[0]
    elements_size = inputs.shape[0]
    width = inputs.shape[1]
    read_head = 0
    data_size = 0

    start = (read_head + data_size) % buffer_size
    r_idx = (read_head + 0) % buffer_size
    off = (r_idx - start) % buffer_size
    hit = off < elements_size

    src = inputs if hit else buffer
    row = off if hit else r_idx
    blk_row, row_in_block = divmod(row, _BLK)

    out = pl.pallas_call(
        lambda src_ref, out_ref: _gather_row_body(row_in_block, src_ref, out_ref),
        grid=(1,),
        in_specs=[pl.BlockSpec((_BLK, width), lambda i: (blk_row, 0))],
        out_specs=pl.BlockSpec((1, width), lambda i: (0, 0)),
        out_shape=jax.ShapeDtypeStruct((1, width), src.dtype),
    )(src)
    return out
